# baseline (speedup 1.0000x reference)
import jax
import jax.numpy as jnp
from jax.experimental import pallas as pl
from jax.experimental.pallas import tpu as pltpu


def _copy_body(idx_ref, x_ref, o_ref):
    o_ref[...] = x_ref[...]


def kernel(x, idxs):
    B, C, T, H, W = x.shape
    BC = B * C
    xr = x.reshape(BC, T, H, W)
    idxs32 = idxs.astype(jnp.int32)

    BC_BLK = 256
    grid = (BC // BC_BLK, T)

    out = pl.pallas_call(
        _copy_body,
        grid_spec=pltpu.PrefetchScalarGridSpec(
            num_scalar_prefetch=1,
            grid=grid,
            in_specs=[
                pl.BlockSpec(
                    (BC_BLK, 1, H, W),
                    lambda i, t, idx_ref: (i, idx_ref[t], 0, 0),
                )
            ],
            out_specs=pl.BlockSpec(
                (BC_BLK, 1, H, W),
                lambda i, t, idx_ref: (i, t, 0, 0),
            ),
        ),
        out_shape=jax.ShapeDtypeStruct((BC, T, H, W), x.dtype),
    )(idxs32, xr)
    return out.reshape(B, C, T, H, W)
